# baseline (speedup 1.0000x reference)
import functools
import jax
import jax.numpy as jnp
from jax.experimental import pallas as pl
from jax.experimental.pallas import tpu as pltpu

B, H, W = 2, 96, 312
C_IMG = 256
HW = H * W
CELLS = B * HW
DUMMY = CELLS
TILE_N = 2048
HT = 24


def _points_body(f_ref, u_ref, v_ref, bidx_ref, ut_ref, p_ref, cell_ref):
    p_ref[...] = jnp.dot(f_ref[...], ut_ref[...],
                         preferred_element_type=jnp.float32)
    uu = jnp.clip(u_ref[...], 0.0, 1.0)
    vv = jnp.clip(v_ref[...], 0.0, 1.0)
    r = jnp.floor(vv * H).astype(jnp.int32)
    c = jnp.floor(uu * W).astype(jnp.int32)
    valid = (r < H) & (c < W)
    cell = (bidx_ref[...] * H + r) * W + c
    cell_ref[...] = jnp.where(valid, cell, DUMMY)


def _project_points(f, u, v, bidx, ut):
    n = f.shape[0]
    nb = n // TILE_N
    cf = f.shape[1]
    grid = (nb,)
    p, cell = pl.pallas_call(
        _points_body,
        grid=grid,
        in_specs=[
            pl.BlockSpec((TILE_N, cf), lambda i: (i, 0)),
            pl.BlockSpec((1, 1, TILE_N), lambda i: (i, 0, 0)),
            pl.BlockSpec((1, 1, TILE_N), lambda i: (i, 0, 0)),
            pl.BlockSpec((1, 1, TILE_N), lambda i: (i, 0, 0)),
            pl.BlockSpec((cf, 16), lambda i: (0, 0)),
        ],
        out_specs=[
            pl.BlockSpec((TILE_N, 16), lambda i: (i, 0)),
            pl.BlockSpec((1, 1, TILE_N), lambda i: (i, 0, 0)),
        ],
        out_shape=[
            jax.ShapeDtypeStruct((n, 16), jnp.float32),
            jax.ShapeDtypeStruct((nb, 1, TILE_N), jnp.int32),
        ],
    )(f, u.reshape(nb, 1, TILE_N), v.reshape(nb, 1, TILE_N),
      bidx.reshape(nb, 1, TILE_N), ut)
    return p, cell.reshape(n)


def _gate_body(x_ref, w_ref, g_ref):
    x = x_ref[0].reshape(C_IMG, HT * W)
    g = jnp.dot(w_ref[...], x, preferred_element_type=jnp.float32)
    g_ref[...] = g.reshape(1, HT, W)


def _gate(x, w):
    return pl.pallas_call(
        _gate_body,
        grid=(B, H // HT),
        in_specs=[
            pl.BlockSpec((1, C_IMG, HT, W), lambda b, j: (b, 0, j, 0)),
            pl.BlockSpec((1, C_IMG), lambda b, j: (0, 0)),
        ],
        out_specs=pl.BlockSpec((1, HT, W), lambda b, j: (b, j, 0)),
        out_shape=jax.ShapeDtypeStruct((B, H, W), jnp.float32),
    )(x, w)


def _fuse_body(x_ref, gk_ref, out_ref):
    j = pl.program_id(1)
    h0 = j * HT
    acc = jnp.zeros((HT, W), jnp.float32)
    for t in range(9):
        dh, dw = t // 3, t % 3
        gt = gk_ref[0, t, pl.ds(h0, HT + 8), :]
        acc = acc + gt[dh:dh + HT, dw:dw + W]
    att = jax.nn.sigmoid(acc)
    out_ref[...] = x_ref[...] * att[None, None]


def _fuse(x, gkp):
    return pl.pallas_call(
        _fuse_body,
        grid=(B, H // HT),
        in_specs=[
            pl.BlockSpec((1, C_IMG, HT, W), lambda b, j: (b, 0, j, 0)),
            pl.BlockSpec((1, 9, H + 8, W + 2), lambda b, j: (b, 0, 0, 0)),
        ],
        out_specs=pl.BlockSpec((1, C_IMG, HT, W), lambda b, j: (b, 0, j, 0)),
        out_shape=jax.ShapeDtypeStruct((B, C_IMG, H, W), jnp.float32),
    )(x, gkp)


def _pad_n(a, n_pad, fill):
    n = a.shape[0]
    pad = [(0, n_pad - n)] + [(0, 0)] * (a.ndim - 1)
    return jnp.pad(a, pad, constant_values=fill)


def kernel(x_rgb0, feats0, feats1, vox3d0, vox3d1, coords0, coords1,
           bidx0, bidx1, Wr0, br0, Wr2, br2, Wr3, br3, Wsp, bsp):
    cr = Wr2.shape[0]
    wsp_flat = Wsp[0].reshape(cr, 9).T
    u_mat = wsp_flat @ (Wr2 @ Wr0)
    v_mat = wsp_flat @ Wr2
    ksum = wsp_flat.sum(axis=1)
    kvec = wsp_flat @ (Wr2 @ br0 + br2) + ksum * br3[0]
    kvec = kvec.at[4].add(bsp[0])
    ut0 = jnp.pad(u_mat.T, ((0, 0), (0, 7)))
    ut1 = jnp.pad(v_mat.T, ((0, 0), (0, 7)))

    n0, n1 = feats0.shape[0], feats1.shape[0]
    n0p = ((n0 + TILE_N - 1) // TILE_N) * TILE_N
    n1p = ((n1 + TILE_N - 1) // TILE_N) * TILE_N
    f0 = _pad_n(jnp.concatenate([feats0, vox3d0], axis=1), n0p, 0.0)
    f1 = _pad_n(jnp.concatenate([feats1, vox3d1], axis=1), n1p, 0.0)
    u0 = _pad_n(coords0[:, 0], n0p, 2.0)
    v0 = _pad_n(coords0[:, 1], n0p, 2.0)
    u1 = _pad_n(coords1[:, 0], n1p, 2.0)
    v1 = _pad_n(coords1[:, 1], n1p, 2.0)
    b0 = _pad_n(bidx0.astype(jnp.int32), n0p, 0)
    b1 = _pad_n(bidx1.astype(jnp.int32), n1p, 0)

    p0, cell0 = _project_points(f0, u0, v0, b0, ut0)
    p1, cell1 = _project_points(f1, u1, v1, b1, ut1)
    graw = _gate(x_rgb0, Wr3)

    def winner_gather(cell, p, npad):
        idx = jnp.arange(npad, dtype=jnp.int32)
        wgrid = jnp.full((CELLS + 1,), -1, jnp.int32).at[cell].max(idx)
        wgrid = wgrid[:CELLS]
        g = p[jnp.maximum(wgrid, 0), :9]
        return jnp.where(wgrid[:, None] >= 0, g, 0.0)

    g0 = winner_gather(cell0, p0, n0p)
    g1 = winner_gather(cell1, p1, n1p)
    gk = g0 + g1 + graw.reshape(CELLS, 1) * ksum[None, :] + kvec[None, :]

    gkp = gk.reshape(B, H, W, 9).transpose(0, 3, 1, 2)
    gkp = jnp.pad(gkp, ((0, 0), (0, 0), (1, 7), (1, 1)))

    return _fuse(x_rgb0, gkp)


# speedup vs baseline: 130.5622x; 130.5622x over previous
import functools
import jax
import jax.numpy as jnp
from jax import lax
from jax.experimental import pallas as pl
from jax.experimental.pallas import tpu as pltpu
from jax.experimental.pallas import tpu_sc as plsc

B, H, W = 2, 96, 312
C_IMG = 256
HW = H * W
CELLS = B * HW
DUMMY = CELLS
TILE_N = 2048
HT = 24

GRID_P = 60928
SLICE = GRID_P // 16
N0P = 151552
N1P = 81920
CH0 = N0P // 16
CH1 = N1P // 16
CPT = CELLS // 32
CPTP = 1920


def _vgather16(x, idx):
    dnums = lax.GatherDimensionNumbers(
        offset_dims=(), collapsed_slice_dims=(0,), start_index_map=(0,))
    return lax.gather(x, idx[:, None], dnums, (1,),
                      indices_are_sorted=False, unique_indices=False,
                      mode=lax.GatherScatterMode.PROMISE_IN_BOUNDS)


def _sc1a_body(cell0, cell1, localw, cellbuf, grid):
    cid = lax.axis_index("c")
    sid = lax.axis_index("s")
    lane = jnp.arange(16, dtype=jnp.int32)
    neg1 = jnp.full((16,), -1, jnp.int32)
    is0 = cid == 0

    def mset(i, _):
        grid[pl.ds(i * 16, 16)] = neg1
        return 0
    lax.fori_loop(0, GRID_P // 16, mset, 0)

    @pl.when(is0)
    def _():
        pltpu.sync_copy(cell0.at[pl.ds(sid * CH0, CH0)], cellbuf)

    @pl.when(jnp.logical_not(is0))
    def _():
        pltpu.sync_copy(cell1.at[pl.ds(sid * CH1, CH1)],
                        cellbuf.at[pl.ds(0, CH1)])

    nvec = jnp.where(is0, CH0 // 16, CH1 // 16)
    base = sid * jnp.where(is0, CH0, CH1)

    def scat(j, _):
        cells = cellbuf[pl.ds(j * 16, 16)]
        key = cells * 16 + lane
        ks, _ = plsc.sort_key_val(key, key)
        cell_s = lax.shift_right_logical(ks, 4)
        lane_s = lax.bitwise_and(ks, 15)
        pidx = base + j * 16 + lane_s
        nxt = _vgather16(cell_s, jnp.minimum(lane + 1, 15))
        lastmask = (cell_s != nxt) | (lane == 15)
        plsc.store_scatter(grid, [cell_s], pidx, mask=lastmask)
        return 0
    lax.fori_loop(0, nvec, scat, 0)

    gbase = (cid * 16 + sid) * GRID_P
    pltpu.sync_copy(grid, localw.at[pl.ds(gbase, GRID_P)])


def _sc1b_body(localw, winner, grid, outbuf):
    cid = lax.axis_index("c")
    sid = lax.axis_index("s")
    cbase = cid * 16 * GRID_P
    for g in range(16):
        pltpu.sync_copy(
            localw.at[pl.ds(cbase + g * GRID_P + sid * SLICE, SLICE)],
            grid.at[pl.ds(g * SLICE, SLICE)])

    def mrg(i, _):
        acc = grid[pl.ds(i * 16, 16)]
        for g in range(1, 16):
            acc = jnp.maximum(acc, grid[pl.ds(g * SLICE + i * 16, 16)])
        outbuf[pl.ds(i * 16, 16)] = acc
        return 0
    lax.fori_loop(0, SLICE // 16, mrg, 0)
    pltpu.sync_copy(outbuf,
                    winner.at[pl.ds(cid * GRID_P + sid * SLICE, SLICE)])


def _sc_winner(cell0, cell1):
    mesh = plsc.VectorSubcoreMesh(core_axis_name="c", subcore_axis_name="s",
                                  num_cores=2, num_subcores=16)
    localw = pl.kernel(
        _sc1a_body,
        out_type=jax.ShapeDtypeStruct((32 * GRID_P,), jnp.int32),
        mesh=mesh,
        compiler_params=pltpu.CompilerParams(needs_layout_passes=False),
        scratch_types=[
            pltpu.VMEM((CH0,), jnp.int32),
            pltpu.VMEM((GRID_P,), jnp.int32),
        ],
    )(cell0, cell1)
    return pl.kernel(
        _sc1b_body,
        out_type=jax.ShapeDtypeStruct((2 * GRID_P,), jnp.int32),
        mesh=mesh,
        compiler_params=pltpu.CompilerParams(needs_layout_passes=False),
        scratch_types=[
            pltpu.VMEM((GRID_P,), jnp.int32),
            pltpu.VMEM((SLICE,), jnp.int32),
        ],
    )(localw)


def _sc2_body(winner, p0, p1, graw, consts, gk,
              w0b, w1b, grawb, val0b, val1b, idx0b, idx1b,
              rows0, rows1, outb, cvec, sem):
    cid = lax.axis_index("c")
    sid = lax.axis_index("s")
    wid = cid * 16 + sid
    b = wid // 16
    hwb = (wid % 16) * CPT
    base = wid * CPT
    lane = jnp.arange(16, dtype=jnp.int32)
    neg1 = jnp.full((16,), -1, jnp.int32)

    pltpu.sync_copy(winner.at[pl.ds(base, CPT)], w0b.at[pl.ds(0, CPT)])
    pltpu.sync_copy(winner.at[pl.ds(GRID_P + base, CPT)],
                    w1b.at[pl.ds(0, CPT)])
    pltpu.sync_copy(graw.at[pl.ds(base, CPT)], grawb.at[pl.ds(0, CPT)])
    pltpu.sync_copy(consts, cvec)
    for k in range(3):
        w0b[pl.ds(CPT + k * 16, 16)] = neg1
        w1b[pl.ds(CPT + k * 16, 16)] = neg1

    def bld(v, _):
        w0 = w0b[pl.ds(v * 16, 16)]
        w1 = w1b[pl.ds(v * 16, 16)]
        spread = lax.bitwise_and(base + v * 16 + lane, 1023)
        val0b[pl.ds(v * 16, 16)] = jnp.where(w0 >= 0, 1.0, 0.0)
        val1b[pl.ds(v * 16, 16)] = jnp.where(w1 >= 0, 1.0, 0.0)
        idx0b[v // 8, pl.ds((v % 8) * 16, 16)] = jnp.where(w0 >= 0, w0, spread)
        idx1b[v // 8, pl.ds((v % 8) * 16, 16)] = jnp.where(w1 >= 0, w1, spread)
        return 0
    lax.fori_loop(0, CPTP // 16, bld, 0)

    for j in range(CPTP // 128):
        c0 = pltpu.async_copy(
            p0.at[idx0b.at[j]], rows0.at[pl.ds(j * 128, 128), :], sem)
        c1 = pltpu.async_copy(
            p1.at[idx1b.at[j]], rows1.at[pl.ds(j * 128, 128), :], sem)
        c0.wait()
        c1.wait()

    def cmb(v, _):
        rowidx = v * 16 + lane
        grawv = grawb[pl.ds(v * 16, 16)]
        v0 = val0b[pl.ds(v * 16, 16)]
        v1 = val1b[pl.ds(v * 16, 16)]
        for t in range(9):
            tcol = jnp.full((16,), t, jnp.int32)
            g0 = plsc.load_gather(rows0, [rowidx, tcol])
            g1 = plsc.load_gather(rows1, [rowidx, tcol])
            res = g0 * v0 + g1 * v1 + grawv * cvec[t] + cvec[16 + t]
            outb[pl.ds(t * CPT + v * 16, 16)] = res
        return 0
    lax.fori_loop(0, CPT // 16, cmb, 0)

    for t in range(9):
        pltpu.sync_copy(outb.at[pl.ds(t * CPT, CPT)],
                        gk.at[pl.ds((b * 9 + t) * HW + hwb, CPT)])


def _sc_gather(winner, p0, p1, graw_flat, consts):
    mesh = plsc.VectorSubcoreMesh(core_axis_name="c", subcore_axis_name="s", num_cores=2, num_subcores=16)
    return pl.kernel(
        _sc2_body,
        out_type=jax.ShapeDtypeStruct((B * 9 * HW,), jnp.float32),
        mesh=mesh,
        compiler_params=pltpu.CompilerParams(needs_layout_passes=False,
                                             use_tc_tiling_on_sc=False),
        scratch_types=[
            pltpu.VMEM((CPTP,), jnp.int32),
            pltpu.VMEM((CPTP,), jnp.int32),
            pltpu.VMEM((CPTP,), jnp.float32),
            pltpu.VMEM((CPTP,), jnp.float32),
            pltpu.VMEM((CPTP,), jnp.float32),
            pltpu.VMEM((CPTP // 128, 128), jnp.int32),
            pltpu.VMEM((CPTP // 128, 128), jnp.int32),
            pltpu.VMEM((CPTP, 16), jnp.float32),
            pltpu.VMEM((CPTP, 16), jnp.float32),
            pltpu.VMEM((9 * CPT,), jnp.float32),
            pltpu.VMEM((32, 16), jnp.float32),
            pltpu.SemaphoreType.DMA,
        ],
    )(winner, p0, p1, graw_flat, consts)


def _points_body(f_ref, u_ref, v_ref, bidx_ref, ut_ref, p_ref, cell_ref):
    p_ref[...] = jnp.dot(f_ref[...], ut_ref[...],
                         preferred_element_type=jnp.float32)
    uu = jnp.clip(u_ref[...], 0.0, 1.0)
    vv = jnp.clip(v_ref[...], 0.0, 1.0)
    r = jnp.floor(vv * H).astype(jnp.int32)
    c = jnp.floor(uu * W).astype(jnp.int32)
    valid = (r < H) & (c < W)
    cell = (bidx_ref[...] * H + r) * W + c
    cell_ref[...] = jnp.where(valid, cell, DUMMY)


def _project_points(f, u, v, bidx, ut):
    n = f.shape[0]
    nb = n // TILE_N
    cf = f.shape[1]
    grid = (nb,)
    p, cell = pl.pallas_call(
        _points_body,
        grid=grid,
        in_specs=[
            pl.BlockSpec((TILE_N, cf), lambda i: (i, 0)),
            pl.BlockSpec((1, 1, TILE_N), lambda i: (i, 0, 0)),
            pl.BlockSpec((1, 1, TILE_N), lambda i: (i, 0, 0)),
            pl.BlockSpec((1, 1, TILE_N), lambda i: (i, 0, 0)),
            pl.BlockSpec((cf, 16), lambda i: (0, 0)),
        ],
        out_specs=[
            pl.BlockSpec((TILE_N, 16), lambda i: (i, 0)),
            pl.BlockSpec((1, 1, TILE_N), lambda i: (i, 0, 0)),
        ],
        out_shape=[
            jax.ShapeDtypeStruct((n, 16), jnp.float32),
            jax.ShapeDtypeStruct((nb, 1, TILE_N), jnp.int32),
        ],
    )(f, u.reshape(nb, 1, TILE_N), v.reshape(nb, 1, TILE_N),
      bidx.reshape(nb, 1, TILE_N), ut)
    return p, cell.reshape(n)


def _gate_body(x_ref, w_ref, g_ref):
    x = x_ref[0].reshape(C_IMG, HT * W)
    g = jnp.dot(w_ref[...], x, preferred_element_type=jnp.float32)
    g_ref[...] = g.reshape(1, HT, W)


def _gate(x, w):
    return pl.pallas_call(
        _gate_body,
        grid=(B, H // HT),
        in_specs=[
            pl.BlockSpec((1, C_IMG, HT, W), lambda b, j: (b, 0, j, 0)),
            pl.BlockSpec((1, C_IMG), lambda b, j: (0, 0)),
        ],
        out_specs=pl.BlockSpec((1, HT, W), lambda b, j: (b, j, 0)),
        out_shape=jax.ShapeDtypeStruct((B, H, W), jnp.float32),
    )(x, w)


def _fuse_body(x_ref, gk_ref, out_ref):
    j = pl.program_id(1)
    h0 = j * HT
    acc = jnp.zeros((HT, W), jnp.float32)
    for t in range(9):
        dh, dw = t // 3, t % 3
        gt = gk_ref[0, t, pl.ds(h0, HT + 8), :]
        acc = acc + gt[dh:dh + HT, dw:dw + W]
    att = jax.nn.sigmoid(acc)
    out_ref[...] = x_ref[...] * att[None, None]


def _fuse(x, gkp):
    return pl.pallas_call(
        _fuse_body,
        grid=(B, H // HT),
        in_specs=[
            pl.BlockSpec((1, C_IMG, HT, W), lambda b, j: (b, 0, j, 0)),
            pl.BlockSpec((1, 9, H + 8, W + 2), lambda b, j: (b, 0, 0, 0)),
        ],
        out_specs=pl.BlockSpec((1, C_IMG, HT, W), lambda b, j: (b, 0, j, 0)),
        out_shape=jax.ShapeDtypeStruct((B, C_IMG, H, W), jnp.float32),
    )(x, gkp)


def _pad_n(a, n_pad, fill):
    n = a.shape[0]
    pad = [(0, n_pad - n)] + [(0, 0)] * (a.ndim - 1)
    return jnp.pad(a, pad, constant_values=fill)


def kernel(x_rgb0, feats0, feats1, vox3d0, vox3d1, coords0, coords1,
           bidx0, bidx1, Wr0, br0, Wr2, br2, Wr3, br3, Wsp, bsp):
    cr = Wr2.shape[0]
    wsp_flat = Wsp[0].reshape(cr, 9).T
    u_mat = wsp_flat @ (Wr2 @ Wr0)
    v_mat = wsp_flat @ Wr2
    ksum = wsp_flat.sum(axis=1)
    kvec = wsp_flat @ (Wr2 @ br0 + br2) + ksum * br3[0]
    kvec = kvec.at[4].add(bsp[0])
    ut0 = jnp.pad(u_mat.T, ((0, 0), (0, 7)))
    ut1 = jnp.pad(v_mat.T, ((0, 0), (0, 7)))

    n0, n1 = feats0.shape[0], feats1.shape[0]
    n0p, n1p = N0P, N1P
    f0 = _pad_n(jnp.concatenate([feats0, vox3d0], axis=1), n0p, 0.0)
    f1 = _pad_n(jnp.concatenate([feats1, vox3d1], axis=1), n1p, 0.0)
    u0 = _pad_n(coords0[:, 0], n0p, 2.0)
    v0 = _pad_n(coords0[:, 1], n0p, 2.0)
    u1 = _pad_n(coords1[:, 0], n1p, 2.0)
    v1 = _pad_n(coords1[:, 1], n1p, 2.0)
    b0 = _pad_n(bidx0.astype(jnp.int32), n0p, 0)
    b1 = _pad_n(bidx1.astype(jnp.int32), n1p, 0)

    p0, cell0 = _project_points(f0, u0, v0, b0, ut0)
    p1, cell1 = _project_points(f1, u1, v1, b1, ut1)
    graw = _gate(x_rgb0, Wr3)

    consts = jnp.zeros((32, 16), jnp.float32)
    consts = consts.at[0:9, :].set(ksum[:, None])
    consts = consts.at[16:25, :].set(kvec[:, None])
    winner = _sc_winner(cell0, cell1)
    gk = _sc_gather(winner, p0, p1, graw.reshape(CELLS), consts)

    gkp = gk.reshape(B, 9, H, W)
    gkp = jnp.pad(gkp, ((0, 0), (0, 0), (1, 7), (1, 1)))

    return _fuse(x_rgb0, gkp)
